# baseline (speedup 1.0000x reference)
import jax
import jax.numpy as jnp
from jax import lax
from jax.experimental import pallas as pl
from jax.experimental.pallas import tpu as pltpu, tpu_sc as plsc

B = 4096
CTX = 77
D = 512
NH = 4
NM = 3
VSLOT = NH + 1
OSLOT = NH + 2 + NM
HEAD = OSLOT + 1
TAIL = CTX - HEAD
LANES = 16
DJ = D // LANES

_info = plsc.get_sparse_core_info()
_NC = _info.num_cores
_NS = _info.num_subcores
NW = _NC * _NS
ROWS_PER_W = B // NW
C = 8
NCHUNK = ROWS_PER_W // C


def _sc_body(tokid_hbm, tokemb_hbm, pos_hbm, ph_hbm, pm_hbm,
             verb_hbm, obj_hbm, vidx_hbm, oidx_hbm, out_hbm,
             base_v, posc_v, pos5_v, pos9_v, head_v, tokid_v,
             vidx_v, oidx_v, vrows_v, orows_v, sem1, sem2, semw):
    wid = lax.axis_index("s") * _NC + lax.axis_index("c")

    pltpu.sync_copy(tokid_hbm, tokid_v)
    pltpu.async_copy(tokemb_hbm.at[tokid_v], base_v.at[0], sem1).wait()
    pltpu.sync_copy(ph_hbm, base_v.at[0, pl.ds(1, NH)])
    pltpu.sync_copy(pm_hbm, base_v.at[0, pl.ds(NH + 2, NM)])
    pltpu.sync_copy(pos_hbm.at[pl.ds(VSLOT, 1)], pos5_v)
    pltpu.sync_copy(pos_hbm.at[pl.ds(OSLOT, 1)], pos9_v)

    for k in range((CTX + 7) // 8):
        n = min(8, CTX - 8 * k)
        pltpu.sync_copy(pos_hbm.at[pl.ds(8 * k, n)], posc_v.at[pl.ds(0, n)])

        def _add_pos(i, carry, k=k):
            for j in range(DJ):
                s = pl.ds(j * LANES, LANES)
                base_v[0, 8 * k + i, s] = base_v[0, 8 * k + i, s] + posc_v[i, s]
            return carry

        lax.fori_loop(0, n, _add_pos, 0)

    def _fill_heads(b, carry):
        def _fill_row(r, rcarry):
            for j in range(DJ):
                s = pl.ds(j * LANES, LANES)
                head_v[b, r, s] = base_v[0, r, s]
            return rcarry

        lax.fori_loop(0, HEAD, _fill_row, 0)
        return carry

    lax.fori_loop(0, C, _fill_heads, 0)

    base_tail = base_v.at[pl.ds(0, 1), pl.ds(HEAD, TAIL)]

    def _chunk(c, carry):
        off = wid * ROWS_PER_W + c * C
        pltpu.sync_copy(vidx_hbm.at[pl.ds(off, C)], vidx_v)
        pltpu.sync_copy(oidx_hbm.at[pl.ds(off, C)], oidx_v)
        cp1 = pltpu.async_copy(verb_hbm.at[vidx_v], vrows_v, sem1)
        cp2 = pltpu.async_copy(obj_hbm.at[oidx_v], orows_v, sem2)
        cp1.wait()
        cp2.wait()

        copies = []
        for i in range(C):
            for j in range(DJ):
                s = pl.ds(j * LANES, LANES)
                head_v[i, VSLOT, s] = vrows_v[i, s] + pos5_v[0, s]
                head_v[i, OSLOT, s] = orows_v[i, s] + pos9_v[0, s]
            row = pl.ds(off + i, 1)
            copies.append(pltpu.async_copy(
                head_v.at[pl.ds(i, 1)], out_hbm.at[row, pl.ds(0, HEAD)], semw))
            copies.append(pltpu.async_copy(
                base_tail, out_hbm.at[row, pl.ds(HEAD, TAIL)], semw))
        for cp in copies:
            cp.wait()
        return carry

    lax.fori_loop(0, NCHUNK, _chunk, 0)


def kernel(pair_idx, token_ids, token_embedding, positional_embedding,
           prompt_vectors_head, prompt_vectors_mid, verb_embedding,
           obj_embedding):
    vidx = pair_idx[:, 0].astype(jnp.int32)
    oidx = pair_idx[:, 1].astype(jnp.int32)
    tokid = token_ids.reshape(CTX).astype(jnp.int32)
    pos = positional_embedding.reshape(CTX, D)
    verb2d = verb_embedding.reshape(-1, D)
    obj2d = obj_embedding.reshape(-1, D)

    mesh = plsc.VectorSubcoreMesh(core_axis_name="c", subcore_axis_name="s")
    f = pl.kernel(
        _sc_body,
        mesh=mesh,
        compiler_params=pltpu.CompilerParams(use_tc_tiling_on_sc=False),
        out_type=jax.ShapeDtypeStruct((B, CTX, D), jnp.float32),
        scratch_types=[
            pltpu.VMEM((1, CTX, D), jnp.float32),
            pltpu.VMEM((8, D), jnp.float32),
            pltpu.VMEM((1, D), jnp.float32),
            pltpu.VMEM((1, D), jnp.float32),
            pltpu.VMEM((C, HEAD, D), jnp.float32),
            pltpu.VMEM((CTX,), jnp.int32),
            pltpu.VMEM((C,), jnp.int32),
            pltpu.VMEM((C,), jnp.int32),
            pltpu.VMEM((C, D), jnp.float32),
            pltpu.VMEM((C, D), jnp.float32),
            pltpu.SemaphoreType.DMA,
            pltpu.SemaphoreType.DMA,
            pltpu.SemaphoreType.DMA,
        ],
    )
    return f(tokid, token_embedding, pos, prompt_vectors_head,
             prompt_vectors_mid, verb2d, obj2d, vidx, oidx)


# speedup vs baseline: 1.8093x; 1.8093x over previous
import jax
import jax.numpy as jnp
from jax import lax
from jax.experimental import pallas as pl
from jax.experimental.pallas import tpu as pltpu, tpu_sc as plsc

B = 4096
CTX = 77
D = 512
NH = 4
NM = 3
VSLOT = NH + 1
OSLOT = NH + 2 + NM
LANES = 16
DJ = D // LANES

_info = plsc.get_sparse_core_info()
_NC = _info.num_cores
_NS = _info.num_subcores
NW = _NC * _NS
ROWS_PER_W = B // NW
C = 32
NCHUNK = ROWS_PER_W // C

G = 32
NG = B // G


def _sc_body(tokid_hbm, tokemb_hbm, ph_hbm, pm_hbm,
             verb_hbm, obj_hbm, vidx_hbm, oidx_hbm,
             baseout_hbm, vout_hbm, oout_hbm,
             base_v, tokid_v, vidx_v, oidx_v, vrows_v, orows_v, sem1, sem2):
    wid = lax.axis_index("s") * _NC + lax.axis_index("c")

    @pl.when(wid == 0)
    def _build_base():
        pltpu.sync_copy(tokid_hbm, tokid_v)
        pltpu.async_copy(tokemb_hbm.at[tokid_v], base_v.at[0], sem1).wait()
        pltpu.sync_copy(ph_hbm, base_v.at[0, pl.ds(1, NH)])
        pltpu.sync_copy(pm_hbm, base_v.at[0, pl.ds(NH + 2, NM)])
        pltpu.sync_copy(base_v.at[0], baseout_hbm)

    def _chunk(c, carry):
        off = wid * ROWS_PER_W + c * C
        pltpu.sync_copy(vidx_hbm.at[pl.ds(off, C)], vidx_v)
        pltpu.sync_copy(oidx_hbm.at[pl.ds(off, C)], oidx_v)
        cp1 = pltpu.async_copy(verb_hbm.at[vidx_v], vrows_v, sem1)
        cp2 = pltpu.async_copy(obj_hbm.at[oidx_v], orows_v, sem2)
        cp1.wait()
        cp2.wait()
        pltpu.sync_copy(vrows_v, vout_hbm.at[pl.ds(off, C)])
        pltpu.sync_copy(orows_v, oout_hbm.at[pl.ds(off, C)])
        return carry

    lax.fori_loop(0, NCHUNK, _chunk, 0)


def _tc_body(base_ref, pos_ref, vrows_ref, orows_ref, out_hbm,
             big0, big1, sem0, sem1):
    g = pl.program_id(0)
    ng = pl.num_programs(0)

    @pl.when(g == 0)
    def _init():
        bp = base_ref[...] + pos_ref[...]
        for i in range(G):
            big0[i] = bp
            big1[i] = bp

    vp = vrows_ref[...] + pos_ref[VSLOT, :][None, :]
    op = orows_ref[...] + pos_ref[OSLOT, :][None, :]

    def _handle(big, sem):
        @pl.when(g >= 2)
        def _wait_prev():
            pltpu.make_async_copy(
                big, out_hbm.at[pl.ds((g - 2) * G, G)], sem).wait()

        big[:, VSLOT, :] = vp
        big[:, OSLOT, :] = op
        pltpu.make_async_copy(big, out_hbm.at[pl.ds(g * G, G)], sem).start()

    @pl.when(g % 2 == 0)
    def _even():
        _handle(big0, sem0)

    @pl.when(g % 2 == 1)
    def _odd():
        _handle(big1, sem1)

    @pl.when(g == ng - 1)
    def _drain():
        pltpu.make_async_copy(big0, out_hbm.at[pl.ds(g * G, G)], sem0).wait()
        pltpu.make_async_copy(big1, out_hbm.at[pl.ds(g * G, G)], sem1).wait()


def kernel(pair_idx, token_ids, token_embedding, positional_embedding,
           prompt_vectors_head, prompt_vectors_mid, verb_embedding,
           obj_embedding):
    vidx = pair_idx[:, 0].astype(jnp.int32)
    oidx = pair_idx[:, 1].astype(jnp.int32)
    tokid = token_ids.reshape(CTX).astype(jnp.int32)
    pos = positional_embedding.reshape(CTX, D)
    verb2d = verb_embedding.reshape(-1, D)
    obj2d = obj_embedding.reshape(-1, D)

    mesh = plsc.VectorSubcoreMesh(core_axis_name="c", subcore_axis_name="s")
    gather = pl.kernel(
        _sc_body,
        mesh=mesh,
        compiler_params=pltpu.CompilerParams(use_tc_tiling_on_sc=False),
        out_type=(
            jax.ShapeDtypeStruct((CTX, D), jnp.float32),
            jax.ShapeDtypeStruct((B, D), jnp.float32),
            jax.ShapeDtypeStruct((B, D), jnp.float32),
        ),
        scratch_types=[
            pltpu.VMEM((1, CTX, D), jnp.float32),
            pltpu.VMEM((CTX,), jnp.int32),
            pltpu.VMEM((C,), jnp.int32),
            pltpu.VMEM((C,), jnp.int32),
            pltpu.VMEM((C, D), jnp.float32),
            pltpu.VMEM((C, D), jnp.float32),
            pltpu.SemaphoreType.DMA,
            pltpu.SemaphoreType.DMA,
        ],
    )
    base77, vrows, orows = gather(tokid, token_embedding,
                                  prompt_vectors_head, prompt_vectors_mid,
                                  verb2d, obj2d, vidx, oidx)

    assemble = pl.pallas_call(
        _tc_body,
        grid=(NG,),
        in_specs=[
            pl.BlockSpec((CTX, D), lambda g: (0, 0)),
            pl.BlockSpec((CTX, D), lambda g: (0, 0)),
            pl.BlockSpec((G, D), lambda g: (g, 0)),
            pl.BlockSpec((G, D), lambda g: (g, 0)),
        ],
        out_specs=pl.BlockSpec(memory_space=pl.ANY),
        out_shape=jax.ShapeDtypeStruct((B, CTX, D), jnp.float32),
        scratch_shapes=[
            pltpu.VMEM((G, CTX, D), jnp.float32),
            pltpu.VMEM((G, CTX, D), jnp.float32),
            pltpu.SemaphoreType.DMA,
            pltpu.SemaphoreType.DMA,
        ],
    )
    return assemble(base77, pos, vrows, orows)


# speedup vs baseline: 2.0735x; 1.1460x over previous
import jax
import jax.numpy as jnp
from jax import lax
from jax.experimental import pallas as pl
from jax.experimental.pallas import tpu as pltpu, tpu_sc as plsc

B = 4096
CTX = 77
D = 512
NH = 4
NM = 3
VSLOT = NH + 1
OSLOT = NH + 2 + NM

_info = plsc.get_sparse_core_info()
_NC = _info.num_cores
_NS = _info.num_subcores
NW = _NC * _NS
ROWS_PER_W = B // NW

G = 32
NG = B // G


def _sc_body(verb_hbm, obj_hbm, vidx_hbm, oidx_hbm, vout_hbm, oout_hbm,
             idx_v, rows_v, sem):
    wid = lax.axis_index("s") * _NC + lax.axis_index("c")
    off = wid * ROWS_PER_W
    pltpu.sync_copy(vidx_hbm.at[pl.ds(off, ROWS_PER_W)], idx_v)
    pltpu.async_copy(verb_hbm.at[idx_v], rows_v, sem).wait()
    pltpu.sync_copy(rows_v, vout_hbm.at[pl.ds(off, ROWS_PER_W)])
    pltpu.sync_copy(oidx_hbm.at[pl.ds(off, ROWS_PER_W)], idx_v)
    pltpu.async_copy(obj_hbm.at[idx_v], rows_v, sem).wait()
    pltpu.sync_copy(rows_v, oout_hbm.at[pl.ds(off, ROWS_PER_W)])


def _tc_body(tokid_sref, tokemb_hbm, pos_ref, ph_ref, pm_ref,
             vrows_ref, orows_ref, out_hbm, basebuf, big0, big1,
             gsem, sem0, sem1):
    g = pl.program_id(0)
    ng = pl.num_programs(0)

    @pl.when(g == 0)
    def _init():
        cps = []
        for r in range(CTX):
            cp = pltpu.make_async_copy(
                tokemb_hbm.at[pl.ds(tokid_sref[r], 1)],
                basebuf.at[pl.ds(r, 1)], gsem)
            cp.start()
            cps.append(cp)
        for cp in cps:
            cp.wait()
        basebuf[1:1 + NH, :] = ph_ref[...]
        basebuf[NH + 2:NH + 2 + NM, :] = pm_ref[...]
        base_val = basebuf[...] + pos_ref[...]
        for i in range(G):
            big0[i] = base_val
            big1[i] = base_val

    vp = vrows_ref[...] + pos_ref[VSLOT, :][None, :]
    op = orows_ref[...] + pos_ref[OSLOT, :][None, :]

    def _handle(big, sem):
        @pl.when(g >= 2)
        def _wait_prev():
            pltpu.make_async_copy(
                big, out_hbm.at[pl.ds((g - 2) * G, G)], sem).wait()

        big[:, VSLOT, :] = vp
        big[:, OSLOT, :] = op
        pltpu.make_async_copy(big, out_hbm.at[pl.ds(g * G, G)], sem).start()

    @pl.when(g % 2 == 0)
    def _even():
        _handle(big0, sem0)

    @pl.when(g % 2 == 1)
    def _odd():
        _handle(big1, sem1)

    @pl.when(g == ng - 1)
    def _drain():
        pltpu.make_async_copy(big0, out_hbm.at[pl.ds(g * G, G)], sem0).wait()
        pltpu.make_async_copy(big1, out_hbm.at[pl.ds(g * G, G)], sem1).wait()


def kernel(pair_idx, token_ids, token_embedding, positional_embedding,
           prompt_vectors_head, prompt_vectors_mid, verb_embedding,
           obj_embedding):
    vidx = pair_idx[:, 0].astype(jnp.int32)
    oidx = pair_idx[:, 1].astype(jnp.int32)
    tokid = token_ids.reshape(CTX).astype(jnp.int32)
    pos = positional_embedding.reshape(CTX, D)
    verb2d = verb_embedding.reshape(-1, D)
    obj2d = obj_embedding.reshape(-1, D)

    mesh = plsc.VectorSubcoreMesh(core_axis_name="c", subcore_axis_name="s")
    gather = pl.kernel(
        _sc_body,
        mesh=mesh,
        out_type=(
            jax.ShapeDtypeStruct((B, D), jnp.float32),
            jax.ShapeDtypeStruct((B, D), jnp.float32),
        ),
        scratch_types=[
            pltpu.VMEM((ROWS_PER_W,), jnp.int32),
            pltpu.VMEM((ROWS_PER_W, D), jnp.float32),
            pltpu.SemaphoreType.DMA,
        ],
    )
    vrows, orows = gather(verb2d, obj2d, vidx, oidx)

    assemble = pl.pallas_call(
        _tc_body,
        grid_spec=pltpu.PrefetchScalarGridSpec(
            num_scalar_prefetch=1,
            grid=(NG,),
            in_specs=[
                pl.BlockSpec(memory_space=pl.ANY),
                pl.BlockSpec((CTX, D), lambda g, s: (0, 0)),
                pl.BlockSpec((NH, D), lambda g, s: (0, 0)),
                pl.BlockSpec((NM, D), lambda g, s: (0, 0)),
                pl.BlockSpec((G, D), lambda g, s: (g, 0)),
                pl.BlockSpec((G, D), lambda g, s: (g, 0)),
            ],
            out_specs=pl.BlockSpec(memory_space=pl.ANY),
            scratch_shapes=[
                pltpu.VMEM((CTX, D), jnp.float32),
                pltpu.VMEM((G, CTX, D), jnp.float32),
                pltpu.VMEM((G, CTX, D), jnp.float32),
                pltpu.SemaphoreType.DMA,
                pltpu.SemaphoreType.DMA,
                pltpu.SemaphoreType.DMA,
            ],
        ),
        out_shape=jax.ShapeDtypeStruct((B, CTX, D), jnp.float32),
    )
    return assemble(tokid, token_embedding, pos, prompt_vectors_head,
                    prompt_vectors_mid, vrows, orows)
